# baseline (speedup 1.0000x reference)
import jax
import jax.numpy as jnp
from jax.experimental import pallas as pl
from jax.experimental.pallas import tpu as pltpu

N_AA = 21
D = 64
LSEQ = 512
KOH = 24
G = 64


def _emb_kernel(idx_ref, phys_ref, T24_ref, W_ref, pos_ref, pm_ref, g_ref,
                b_ref, ones_ref, out_ref):
    T24 = T24_ref[...]
    W = W_ref[...]
    posb = pos_ref[...]
    pmean = pm_ref[...]
    gT = g_ref[...]
    bT = b_ref[...]
    dn = (((0,), (0,)), ((), ()))
    for g in range(G):
        idx = idx_ref[g:g + 1, :]
        s = jax.lax.broadcasted_iota(jnp.int32, (KOH, LSEQ), 0)
        oh = (s == idx).astype(jnp.float32)
        ext = jax.lax.dot_general(T24, oh, dn,
                                  preferred_element_type=jnp.float32)
        ext = ext + jax.lax.dot_general(W, phys_ref[g], dn,
                                        preferred_element_type=jnp.float32)
        emb = ext[:D] + posb
        mean = ext[D:D + 1] + pmean
        c = emb - mean
        var = jax.lax.dot_general(ones_ref[...], c * c, dn,
                                  preferred_element_type=jnp.float32)
        inv = jax.lax.rsqrt(var + 1e-5)
        out_ref[g] = c * inv * gT + bT


@jax.jit
def kernel(aa_idx, physchem, aa_table, W_phys, b_phys, pos_table, gamma, beta):
    Bsz, Ls = aa_idx.shape
    T24 = (jnp.zeros((KOH, D + 8), jnp.float32)
           .at[:N_AA, :D].set(aa_table)
           .at[:N_AA, D].set(jnp.sum(aa_table, axis=1) / D))
    Wx = (jnp.zeros((3, D + 8), jnp.float32)
          .at[:, :D].set(W_phys)
          .at[:, D].set(jnp.sum(W_phys, axis=1) / D))
    posb = pos_table.T + b_phys[:, None]
    pmean = jnp.mean(posb, axis=0, keepdims=True)
    gT = gamma.reshape(D, 1)
    bT = beta.reshape(D, 1)

    out = pl.pallas_call(
        _emb_kernel,
        grid=(Bsz // G,),
        in_specs=[
            pl.BlockSpec((G, LSEQ), lambda i: (i, 0)),
            pl.BlockSpec((G, 3, LSEQ), lambda i: (i, 0, 0)),
            pl.BlockSpec((KOH, D + 8), lambda i: (0, 0)),
            pl.BlockSpec((3, D + 8), lambda i: (0, 0)),
            pl.BlockSpec((D, LSEQ), lambda i: (0, 0)),
            pl.BlockSpec((1, LSEQ), lambda i: (0, 0)),
            pl.BlockSpec((D, 1), lambda i: (0, 0)),
            pl.BlockSpec((D, 1), lambda i: (0, 0)),
            pl.BlockSpec((D, 1), lambda i: (0, 0)),
        ],
        out_specs=pl.BlockSpec((G, D, LSEQ), lambda i: (i, 0, 0)),
        out_shape=jax.ShapeDtypeStruct((Bsz, D, LSEQ), jnp.float32),
        compiler_params=pltpu.CompilerParams(
            dimension_semantics=("arbitrary",),
        ),
    )(aa_idx.astype(jnp.int32), physchem.transpose(0, 2, 1), T24, Wx,
      posb, pmean, gT, bT, jnp.full((D, 1), 1.0 / D, jnp.float32))
    return out.transpose(0, 2, 1)


# speedup vs baseline: 2.5369x; 2.5369x over previous
import jax
import jax.numpy as jnp
from jax.experimental import pallas as pl
from jax.experimental.pallas import tpu as pltpu

N_AA = 21
D = 64
LSEQ = 512
KOH = 24
G = 64


def _emb_kernel(idx_ref, phys_ref, T24_ref, W_ref, pos_ref, pm_ref, g_ref,
                b_ref, out_ref):
    T24 = T24_ref[...]
    W = W_ref[...]
    posb = pos_ref[...]
    pmean = pm_ref[...]
    gT = g_ref[...]
    bT = b_ref[...]
    dn = (((0,), (0,)), ((), ()))
    for g in range(G):
        idx = idx_ref[g:g + 1, :]
        s = jax.lax.broadcasted_iota(jnp.int32, (KOH, LSEQ), 0)
        oh = (s == idx).astype(jnp.float32)
        ext = jax.lax.dot_general(T24, oh, dn,
                                  preferred_element_type=jnp.float32)
        ext = ext + jax.lax.dot_general(W, phys_ref[g], dn,
                                        preferred_element_type=jnp.float32)
        emb = ext[:D] + posb
        mean = ext[D:D + 1] + pmean
        msq = jnp.mean(emb * emb, axis=0, keepdims=True)
        var = msq - mean * mean
        inv = jax.lax.rsqrt(var + 1e-5)
        out_ref[g] = (emb - mean) * inv * gT + bT


@jax.jit
def kernel(aa_idx, physchem, aa_table, W_phys, b_phys, pos_table, gamma, beta):
    Bsz, Ls = aa_idx.shape
    T24 = (jnp.zeros((KOH, D + 8), jnp.float32)
           .at[:N_AA, :D].set(aa_table)
           .at[:N_AA, D].set(jnp.sum(aa_table, axis=1) / D))
    Wx = (jnp.zeros((3, D + 8), jnp.float32)
          .at[:, :D].set(W_phys)
          .at[:, D].set(jnp.sum(W_phys, axis=1) / D))
    posb = pos_table.T + b_phys[:, None]
    pmean = jnp.mean(posb, axis=0, keepdims=True)
    gT = gamma.reshape(D, 1)
    bT = beta.reshape(D, 1)

    out = pl.pallas_call(
        _emb_kernel,
        grid=(Bsz // G,),
        in_specs=[
            pl.BlockSpec((G, LSEQ), lambda i: (i, 0)),
            pl.BlockSpec((G, 3, LSEQ), lambda i: (i, 0, 0)),
            pl.BlockSpec((KOH, D + 8), lambda i: (0, 0)),
            pl.BlockSpec((3, D + 8), lambda i: (0, 0)),
            pl.BlockSpec((D, LSEQ), lambda i: (0, 0)),
            pl.BlockSpec((1, LSEQ), lambda i: (0, 0)),
            pl.BlockSpec((D, 1), lambda i: (0, 0)),
            pl.BlockSpec((D, 1), lambda i: (0, 0)),
        ],
        out_specs=pl.BlockSpec((G, D, LSEQ), lambda i: (i, 0, 0)),
        out_shape=jax.ShapeDtypeStruct((Bsz, D, LSEQ), jnp.float32),
        compiler_params=pltpu.CompilerParams(
            dimension_semantics=("arbitrary",),
        ),
    )(aa_idx.astype(jnp.int32), physchem.transpose(0, 2, 1), T24, Wx,
      posb, pmean, gT, bT)
    return out.transpose(0, 2, 1)
